# Initial kernel scaffold; baseline (speedup 1.0000x reference)
import jax
import jax.numpy as jnp
from jax import lax
from jax.experimental import pallas as pl
from jax.experimental.pallas import tpu as pltpu
from jax.experimental.pallas import tpu_sc as plsc

N = 100000
F = 26
K = 16
BATCH = 4096
D = F * K
TOTAL = BATCH * F

_info = plsc.get_sparse_core_info()
NC = _info.num_cores
NS = _info.num_subcores
NW = NC * NS
PER_W = TOTAL // NW
CHUNK = 128
NCHUNK = PER_W // CHUNK


def _gather_body(table_hbm, idx_hbm, out_hbm,
                 idx_v, buf0, buf1, gsem0, gsem1, ssem0, ssem1):
    wid = lax.axis_index("s") * NC + lax.axis_index("c")
    base = wid * PER_W
    pltpu.sync_copy(idx_hbm.at[wid], idx_v)

    bufs = (buf0, buf1)
    gsems = (gsem0, gsem1)
    ssems = (ssem0, ssem1)

    def wait_store(b):
        pltpu.make_async_copy(
            bufs[b], out_hbm.at[pl.ds(base, CHUNK)], ssems[b]).wait()

    def wait_gather(b):
        pltpu.make_async_copy(
            table_hbm.at[idx_v.at[0]], bufs[b], gsems[b]).wait()

    pltpu.async_copy(table_hbm.at[idx_v.at[0]], buf0, gsem0)

    @pl.loop(0, NCHUNK, step=2)
    def _(g):
        for b in range(2):
            c = g + b
            nb = 1 - b
            cn = c + 1

            @pl.when(cn < NCHUNK)
            def _():
                @pl.when(cn >= 2)
                def _():
                    wait_store(nb)
                pltpu.async_copy(table_hbm.at[idx_v.at[cn]], bufs[nb],
                                 gsems[nb])

            wait_gather(b)
            pltpu.async_copy(
                bufs[b], out_hbm.at[pl.ds(base + c * CHUNK, CHUNK)], ssems[b])

    wait_store(0)
    wait_store(1)


@jax.jit
def _embedding3d(idx3, table):
    mesh = plsc.VectorSubcoreMesh(core_axis_name="c", subcore_axis_name="s")
    k = pl.kernel(
        _gather_body,
        out_type=jax.ShapeDtypeStruct((TOTAL, D), jnp.float32),
        mesh=mesh,
        scratch_types=[
            pltpu.VMEM((NCHUNK, CHUNK), jnp.int32),
            pltpu.VMEM((CHUNK, D), jnp.float32),
            pltpu.VMEM((CHUNK, D), jnp.float32),
            pltpu.SemaphoreType.DMA,
            pltpu.SemaphoreType.DMA,
            pltpu.SemaphoreType.DMA,
            pltpu.SemaphoreType.DMA,
        ],
    )
    return k(table, idx3)


def kernel(input, weight):
    idx3 = input.reshape(-1).astype(jnp.int32).reshape(NW, NCHUNK, CHUNK)
    table = weight.reshape(N, D)
    out = _embedding3d(idx3, table)
    return out.reshape(BATCH, F, F, K)


# speedup vs baseline: 5.1802x; 5.1802x over previous
import jax
import jax.numpy as jnp
from jax import lax
from jax.experimental import pallas as pl
from jax.experimental.pallas import tpu as pltpu
from jax.experimental.pallas import tpu_sc as plsc

N = 100000
F = 26
K = 16
BATCH = 4096
D = F * K
TOTAL = BATCH * F

NC = 2
NS = 16
NW = NC * NS
PER_W = TOTAL // NW
CHUNK = 128
NCHUNK = PER_W // CHUNK


def _gather_body(table_hbm, idx_hbm, out_hbm,
                 idx_v, buf0, buf1, gsem0, gsem1, ssem0, ssem1):
    wid = lax.axis_index("s") * NC + lax.axis_index("c")
    base = wid * PER_W
    pltpu.sync_copy(idx_hbm.at[wid], idx_v)

    bufs = (buf0, buf1)
    gsems = (gsem0, gsem1)
    ssems = (ssem0, ssem1)

    def wait_store(b):
        pltpu.make_async_copy(
            bufs[b], out_hbm.at[pl.ds(base, CHUNK)], ssems[b]).wait()

    def wait_gather(b):
        pltpu.make_async_copy(
            table_hbm.at[idx_v.at[0]], bufs[b], gsems[b]).wait()

    pltpu.async_copy(table_hbm.at[idx_v.at[0]], buf0, gsem0)

    @pl.loop(0, NCHUNK, step=2)
    def _(g):
        for b in range(2):
            c = g + b
            nb = 1 - b
            cn = c + 1

            @pl.when(cn < NCHUNK)
            def _():
                @pl.when(cn >= 2)
                def _():
                    wait_store(nb)
                pltpu.async_copy(table_hbm.at[idx_v.at[cn]], bufs[nb],
                                 gsems[nb])

            wait_gather(b)
            pltpu.async_copy(
                bufs[b], out_hbm.at[pl.ds(base + c * CHUNK, CHUNK)], ssems[b])

    wait_store(0)
    wait_store(1)


@jax.jit
def _embedding3d(idx3, table):
    mesh = plsc.VectorSubcoreMesh(core_axis_name="c", subcore_axis_name="s")
    k = pl.kernel(
        _gather_body,
        out_type=jax.ShapeDtypeStruct((TOTAL, D), jnp.float32),
        mesh=mesh,
        compiler_params=pltpu.CompilerParams(use_tc_tiling_on_sc=False),
        scratch_types=[
            pltpu.VMEM((NCHUNK, CHUNK), jnp.int32),
            pltpu.VMEM((CHUNK, D), jnp.float32),
            pltpu.VMEM((CHUNK, D), jnp.float32),
            pltpu.SemaphoreType.DMA,
            pltpu.SemaphoreType.DMA,
            pltpu.SemaphoreType.DMA,
            pltpu.SemaphoreType.DMA,
        ],
    )
    return k(table, idx3)


def kernel(input, weight):
    idx3 = input.reshape(-1).astype(jnp.int32).reshape(NW, NCHUNK, CHUNK)
    table = weight.reshape(N, D)
    out = _embedding3d(idx3, table)
    return out.reshape(BATCH, F, F, K)


# speedup vs baseline: 5.8398x; 1.1273x over previous
import jax
import jax.numpy as jnp
from jax import lax
from jax.experimental import pallas as pl
from jax.experimental.pallas import tpu as pltpu
from jax.experimental.pallas import tpu_sc as plsc

N = 100000
F = 26
K = 16
BATCH = 4096
D = F * K
TOTAL = BATCH * F

NC = 2
NS = 16
NW = NC * NS
PER_W = TOTAL // NW
CHUNK = 128
NCHUNK = PER_W // CHUNK


def _gather_body(table_hbm, idx_hbm, out_hbm,
                 idx_v, buf0, buf1, gsem0, gsem1, ssem0, ssem1):
    wid = lax.axis_index("s") * NC + lax.axis_index("c")
    base = wid * PER_W
    pltpu.sync_copy(idx_hbm.at[wid], idx_v)

    bufs = (buf0, buf1)
    gsems = (gsem0, gsem1)
    ssems = (ssem0, ssem1)

    def wait_store(b):
        pltpu.make_async_copy(
            bufs[b], out_hbm.at[pl.ds(base, CHUNK)], ssems[b]).wait()

    def wait_gather(b):
        pltpu.make_async_copy(
            table_hbm.at[idx_v.at[0]], bufs[b], gsems[b]).wait()

    pltpu.async_copy(table_hbm.at[idx_v.at[0]], buf0, gsem0)

    @pl.loop(0, NCHUNK, step=2)
    def _(g):
        for b in range(2):
            c = g + b
            nb = 1 - b
            cn = c + 1

            @pl.when(cn < NCHUNK)
            def _():
                @pl.when(cn >= 2)
                def _():
                    wait_store(nb)
                pltpu.async_copy(table_hbm.at[idx_v.at[cn]], bufs[nb],
                                 gsems[nb])

            wait_gather(b)
            pltpu.async_copy(
                bufs[b], out_hbm.at[pl.ds(base + c * CHUNK, CHUNK)], ssems[b])

    wait_store(0)
    wait_store(1)


@jax.jit
def _embedding3d(idx3, table):
    mesh = plsc.VectorSubcoreMesh(core_axis_name="c", subcore_axis_name="s")
    k = pl.kernel(
        _gather_body,
        out_type=jax.ShapeDtypeStruct((TOTAL, F, K), jnp.float32),
        mesh=mesh,
        compiler_params=pltpu.CompilerParams(use_tc_tiling_on_sc=False),
        scratch_types=[
            pltpu.VMEM((NCHUNK, CHUNK), jnp.int32),
            pltpu.VMEM((CHUNK, F, K), jnp.float32),
            pltpu.VMEM((CHUNK, F, K), jnp.float32),
            pltpu.SemaphoreType.DMA,
            pltpu.SemaphoreType.DMA,
            pltpu.SemaphoreType.DMA,
            pltpu.SemaphoreType.DMA,
        ],
    )
    return k(table, idx3)


def kernel(input, weight):
    idx3 = input.reshape(-1).astype(jnp.int32).reshape(NW, NCHUNK, CHUNK)
    out = _embedding3d(idx3, weight)
    return out.reshape(BATCH, F, F, K)
